# baseline (speedup 1.0000x reference)
import jax
import jax.numpy as jnp
from jax.experimental import pallas as pl
from jax.experimental.pallas import tpu as pltpu

BM = 200
NBUF = 4


def _leaky(x):
    return jnp.where(x >= 0, x, 0.01 * x)


def _xwt(x, w):
    return jax.lax.dot_general(
        x, w, (((1,), (1,)), ((), ())), preferred_element_type=jnp.float32
    )


def _outer(a_hbm, ego_ref, w1_ref, b1_ref, w2_ref, b2_ref, out_hbm,
           ego_bf_ref):
    n, d = ego_ref.shape
    nm = n // BM
    ego_bf_ref[...] = ego_ref[...].astype(jnp.bfloat16)

    def inner(a_ref, egorow_ref, out_ref):
        a_bf = a_ref[...].astype(jnp.bfloat16)
        side = jnp.dot(a_bf, ego_bf_ref[...],
                       preferred_element_type=jnp.float32)
        er = egorow_ref[...]
        sum_e = _leaky(_xwt(er + side, w1_ref[...]) + b1_ref[...])
        bi_e = _leaky(_xwt(er * side, w2_ref[...]) + b2_ref[...])
        out_ref[...] = sum_e + bi_e

    pipe = pltpu.emit_pipeline(
        inner,
        grid=(nm,),
        in_specs=[
            pl.BlockSpec((BM, n), lambda i: (i, 0),
                         pipeline_mode=pl.Buffered(buffer_count=NBUF)),
            pl.BlockSpec((BM, d), lambda i: (i, 0)),
        ],
        out_specs=[pl.BlockSpec((BM, d), lambda i: (i, 0))],
    )
    pipe(a_hbm, ego_ref, out_hbm)


@jax.jit
def kernel(ego_embeddings, A_in, W1, b1, W2, b2):
    n, d = ego_embeddings.shape
    b1r = b1.reshape(1, d)
    b2r = b2.reshape(1, d)

    out = pl.pallas_call(
        _outer,
        in_specs=[
            pl.BlockSpec(memory_space=pl.ANY),
            pl.BlockSpec((n, d), lambda: (0, 0)),
            pl.BlockSpec((d, d), lambda: (0, 0)),
            pl.BlockSpec((1, d), lambda: (0, 0)),
            pl.BlockSpec((d, d), lambda: (0, 0)),
            pl.BlockSpec((1, d), lambda: (0, 0)),
        ],
        out_specs=pl.BlockSpec(memory_space=pl.ANY),
        out_shape=jax.ShapeDtypeStruct((n, d), jnp.float32),
        scratch_shapes=[pltpu.VMEM((n, d), jnp.bfloat16)],
    )(A_in, ego_embeddings, W1, b1r, W2, b2r)
    return out
